# Initial kernel scaffold; baseline (speedup 1.0000x reference)
import functools

import jax
import jax.numpy as jnp
from jax import lax
from jax.experimental import pallas as pl
from jax.experimental.pallas import tpu as pltpu
from jax.experimental.pallas import tpu_sc as plsc


def _make_gather(n: int, n_ch: int, ch: int, d: int):
    info = plsc.get_sparse_core_info()
    nc, ns = info.num_cores, info.num_subcores
    mesh = plsc.VectorSubcoreMesh(core_axis_name="c", subcore_axis_name="s")

    @functools.partial(
        pl.kernel,
        mesh=mesh,
        out_type=jax.ShapeDtypeStruct((n, d), jnp.float32),
        scratch_types=[
            pltpu.VMEM((n_ch, ch), jnp.int32),
            pltpu.VMEM((ch, d), jnp.float32),
            pltpu.SemaphoreType.DMA,
        ],
    )
    def k(idx_hbm, table_hbm, out_hbm, idx_v, rows_v, sem):
        wid = lax.axis_index("s") * nc + lax.axis_index("c")
        pltpu.sync_copy(idx_hbm.at[pl.ds(wid * n_ch, n_ch)], idx_v)

        def body(j, carry):
            pltpu.async_copy(table_hbm.at[idx_v.at[j]], rows_v, sem).wait()
            pltpu.sync_copy(
                rows_v, out_hbm.at[pl.ds((wid * n_ch + j) * ch, ch)]
            )
            return carry

        lax.fori_loop(0, n_ch, body, 0)

    return k


def kernel(indices, table):
    b, l = indices.shape
    v, d = table.shape
    n = b * l
    ch = 128
    nw = 32
    n_ch = n // (nw * ch)
    idx2d = indices.reshape(nw * n_ch, ch).astype(jnp.int32)
    gather = _make_gather(n, n_ch, ch, d)
    out = gather(idx2d, table)
    return out.reshape(b, l, d)


# speedup vs baseline: 1.6834x; 1.6834x over previous
import functools

import jax
import jax.numpy as jnp
from jax import lax
from jax.experimental import pallas as pl
from jax.experimental.pallas import tpu as pltpu
from jax.experimental.pallas import tpu_sc as plsc


def _make_gather(n: int, n_ch: int, ch: int, d: int):
    info = plsc.get_sparse_core_info()
    nc, ns = info.num_cores, info.num_subcores
    mesh = plsc.VectorSubcoreMesh(core_axis_name="c", subcore_axis_name="s")

    @functools.partial(
        pl.kernel,
        mesh=mesh,
        out_type=jax.ShapeDtypeStruct((n, d), jnp.float32),
        scratch_types=[
            pltpu.VMEM((n_ch, ch), jnp.int32),
            pltpu.VMEM((ch, d), jnp.float32),
            pltpu.SemaphoreType.DMA,
        ],
        compiler_params=pltpu.CompilerParams(use_tc_tiling_on_sc=False),
    )
    def k(idx_hbm, table_hbm, out_hbm, idx_v, rows_v, sem):
        wid = lax.axis_index("s") * nc + lax.axis_index("c")
        pltpu.sync_copy(idx_hbm.at[pl.ds(wid * n_ch, n_ch)], idx_v)

        def body(j, carry):
            pltpu.async_copy(table_hbm.at[idx_v.at[j]], rows_v, sem).wait()
            pltpu.sync_copy(
                rows_v, out_hbm.at[pl.ds((wid * n_ch + j) * ch, ch)]
            )
            return carry

        lax.fori_loop(0, n_ch, body, 0)

    return k


def kernel(indices, table):
    b, l = indices.shape
    v, d = table.shape
    n = b * l
    ch = 128
    nw = 32
    n_ch = n // (nw * ch)
    idx2d = indices.reshape(nw * n_ch, ch).astype(jnp.int32)
    gather = _make_gather(n, n_ch, ch, d)
    out = gather(idx2d, table)
    return out.reshape(b, l, d)


# speedup vs baseline: 1.8709x; 1.1114x over previous
import functools

import jax
import jax.numpy as jnp
from jax import lax
from jax.experimental import pallas as pl
from jax.experimental.pallas import tpu as pltpu
from jax.experimental.pallas import tpu_sc as plsc

_CH = 128
_G = 4
_NSLOT = 2


def _make_gather(n: int, n_ch: int, d: int):
    info = plsc.get_sparse_core_info()
    nc, ns = info.num_cores, info.num_subcores
    mesh = plsc.VectorSubcoreMesh(core_axis_name="c", subcore_axis_name="s")
    n_grp = n_ch // _G
    grp_rows = _G * _CH

    @functools.partial(
        pl.kernel,
        mesh=mesh,
        out_type=jax.ShapeDtypeStruct((n, d), jnp.float32),
        scratch_types=[
            pltpu.VMEM((n_ch, _CH), jnp.int32),
            pltpu.VMEM((_NSLOT, grp_rows, d), jnp.float32),
            [pltpu.SemaphoreType.DMA] * _NSLOT,
            [pltpu.SemaphoreType.DMA] * _NSLOT,
        ],
        compiler_params=pltpu.CompilerParams(use_tc_tiling_on_sc=False),
    )
    def k(idx_hbm, table_hbm, out_hbm, idx_v, rows_v, gsems, osems):
        wid = lax.axis_index("s") * nc + lax.axis_index("c")
        pltpu.sync_copy(idx_hbm.at[pl.ds(wid * n_ch, n_ch)], idx_v)

        def step(st, carry):
            for s in range(_NSLOT):
                g = st * _NSLOT + s
                slot = rows_v.at[s]
                @pl.when(st >= 1)
                def _():
                    pltpu.make_async_copy(
                        out_hbm.at[pl.ds(0, grp_rows)], slot, osems[s]
                    ).wait()

                handles = [
                    pltpu.async_copy(
                        table_hbm.at[idx_v.at[g * _G + q]],
                        slot.at[pl.ds(q * _CH, _CH)],
                        gsems[s],
                    )
                    for q in range(_G)
                ]
                for h in handles:
                    h.wait()
                pltpu.async_copy(
                    slot,
                    out_hbm.at[pl.ds((wid * n_ch + g * _G) * _CH, grp_rows)],
                    osems[s],
                )
            return carry

        lax.fori_loop(0, n_grp // _NSLOT, step, 0)
        for s in range(_NSLOT):
            pltpu.make_async_copy(
                out_hbm.at[pl.ds(0, grp_rows)], rows_v.at[s], osems[s]
            ).wait()

    return k


def kernel(indices, table):
    b, l = indices.shape
    v, d = table.shape
    n = b * l
    nw = 32
    n_ch = n // (nw * _CH)
    idx2d = indices.reshape(nw * n_ch, _CH).astype(jnp.int32)
    gather = _make_gather(n, n_ch, d)
    out = gather(idx2d, table)
    return out.reshape(b, l, d)


# speedup vs baseline: 1.8777x; 1.0037x over previous
import functools

import jax
import jax.numpy as jnp
from jax import lax
from jax.experimental import pallas as pl
from jax.experimental.pallas import tpu as pltpu
from jax.experimental.pallas import tpu_sc as plsc

_CH = 128
_G = 5
_NSLOT = 2


def _make_gather(n: int, n_ch: int, d: int):
    info = plsc.get_sparse_core_info()
    nc, ns = info.num_cores, info.num_subcores
    mesh = plsc.VectorSubcoreMesh(core_axis_name="c", subcore_axis_name="s")
    n_grp = n_ch // _G
    grp_rows = _G * _CH

    @functools.partial(
        pl.kernel,
        mesh=mesh,
        out_type=jax.ShapeDtypeStruct((n, d), jnp.float32),
        scratch_types=[
            pltpu.VMEM((n_ch, _CH), jnp.int32),
            pltpu.VMEM((_NSLOT, grp_rows, d), jnp.float32),
            [pltpu.SemaphoreType.DMA] * _NSLOT,
            [pltpu.SemaphoreType.DMA] * _NSLOT,
        ],
        compiler_params=pltpu.CompilerParams(use_tc_tiling_on_sc=False),
    )
    def k(idx_hbm, table_hbm, out_hbm, idx_v, rows_v, gsems, osems):
        wid = lax.axis_index("s") * nc + lax.axis_index("c")
        pltpu.sync_copy(idx_hbm.at[pl.ds(wid * n_ch, n_ch)], idx_v)

        def fire(g, s):
            for q in range(_G):
                pltpu.async_copy(
                    table_hbm.at[idx_v.at[g * _G + q]],
                    rows_v.at[s].at[pl.ds(q * _CH, _CH)],
                    gsems[s],
                )

        def drain(ref, sem):
            pltpu.make_async_copy(
                out_hbm.at[pl.ds(0, grp_rows)], ref, sem
            ).wait()

        fire(0, 0)

        def step(r, carry):
            for s in range(_NSLOT):
                g = r * _NSLOT + s
                s2 = (s + 1) % _NSLOT

                @pl.when(g + 1 < n_grp)
                def _():
                    @pl.when(g + 1 >= _NSLOT)
                    def _():
                        drain(rows_v.at[s2], osems[s2])

                    fire(g + 1, s2)

                drain(rows_v.at[s], gsems[s])
                pltpu.async_copy(
                    rows_v.at[s],
                    out_hbm.at[pl.ds((wid * n_ch + g * _G) * _CH, grp_rows)],
                    osems[s],
                )
            return carry

        lax.fori_loop(0, n_grp // _NSLOT, step, 0)
        for s in range(_NSLOT):
            drain(rows_v.at[s], osems[s])

    return k


def kernel(indices, table):
    b, l = indices.shape
    v, d = table.shape
    n = b * l
    nw = 32
    n_ch = n // (nw * _CH)
    idx2d = indices.reshape(nw * n_ch, _CH).astype(jnp.int32)
    gather = _make_gather(n, n_ch, d)
    out = gather(idx2d, table)
    return out.reshape(b, l, d)
